# Initial kernel scaffold; baseline (speedup 1.0000x reference)
import jax
import jax.numpy as jnp
from jax.experimental import pallas as pl


def kernel(xs, gs, enc_W0, enc_b0, enc_W1, enc_b1, enc_W2, enc_b2, dec_W0, dec_W1, dec_W2):
    raise NotImplementedError("write your pallas kernel here")


# speedup vs baseline: 6.8296x; 6.8296x over previous
import functools

import jax
import jax.numpy as jnp
from jax.experimental import pallas as pl
from jax.experimental.pallas import tpu as pltpu

D = 3
N = 4096
K = 32
NH = 64
WIDTH = 1.0

QBLK = 256


def _tc_body(xsgs_ref, q_ref, xsT_ref, w0_ref, b0_ref, w1_ref, b1_ref,
             w2_ref, b2_ref, d0_ref, d1_ref, d2w_ref, out_ref,
             dist_ref, inp_ref):
    q = q_ref[...]
    d2 = jnp.zeros((QBLK, N), jnp.float32)
    for d in range(D):
        qd = q[:, d:d + 1]
        pt = xsT_ref[d:d + 1, :]
        dif = qd - pt
        w = dif - WIDTH * jnp.round(dif / WIDTH)
        d2 = d2 + w * w
    dist_ref[...] = d2

    lane_iota = jax.lax.broadcasted_iota(jnp.int32, (QBLK, N), 1)
    BIG = jnp.int32(2**31 - 1)

    def body(k, carry):
        d2c = dist_ref[...]
        m = jnp.min(d2c, axis=1, keepdims=True)
        eq = d2c == m
        idx = jnp.min(jnp.where(eq, lane_iota, BIG), axis=1, keepdims=True)
        onehot = lane_iota == idx
        oh = onehot.astype(jnp.float32)
        nbr = jax.lax.dot_general(oh, xsgs_ref[...],
                                  (((1,), (0,)), ((), ())),
                                  preferred_element_type=jnp.float32)
        inp_ref[pl.ds(k, 1)] = nbr.reshape(1, QBLK, 8)
        dist_ref[...] = jnp.where(onehot, jnp.inf, d2c)
        return carry

    jax.lax.fori_loop(0, K, body, 0, unroll=False)

    nbrs = inp_ref[...]
    dif = q[None, :, :] - nbrs
    colmask = (jax.lax.broadcasted_iota(jnp.int32, (1, 1, 8), 2) < D
               ).astype(jnp.float32)
    inp = dif - colmask * (WIDTH * jnp.round(dif / WIDTH))
    rows = inp.reshape(K * QBLK, 8)

    mm = functools.partial(jax.lax.dot_general,
                           dimension_numbers=(((1,), (0,)), ((), ())),
                           preferred_element_type=jnp.float32)
    gelu = functools.partial(jax.nn.gelu, approximate=True)

    h = gelu(mm(rows, w0_ref[...]) + b0_ref[...])
    h = gelu(mm(h, w1_ref[...]) + b1_ref[...])
    h = mm(h, w2_ref[...])
    pooled = jnp.mean(h.reshape(K, QBLK, NH), axis=0) + b2_ref[...]
    h = gelu(mm(pooled, d0_ref[...]))
    h = gelu(mm(h, d1_ref[...]))
    out_ref[...] = mm(h, d2w_ref[...])


def kernel(xs, gs, enc_W0, enc_b0, enc_W1, enc_b1, enc_W2, enc_b2,
           dec_W0, dec_W1, dec_W2):
    xsgs = jnp.concatenate(
        [xs, gs, jnp.zeros((N, 2), jnp.float32)], axis=1)
    xsT = xs.T
    W0p = jnp.zeros((8, NH), jnp.float32).at[:2 * D].set(enc_W0)

    grid = (N // QBLK,)
    full = lambda shape: pl.BlockSpec(shape, lambda i: (0,) * len(shape))
    out = pl.pallas_call(
        _tc_body,
        grid=grid,
        in_specs=[
            full((N, 8)),
            pl.BlockSpec((QBLK, 8), lambda i: (i, 0)),
            full((D, N)),
            full((8, NH)), full((1, NH)),
            full((NH, NH)), full((1, NH)),
            full((NH, NH)), full((1, NH)),
            full((NH, NH)), full((NH, NH)), full((NH, D)),
        ],
        out_specs=pl.BlockSpec((QBLK, D), lambda i: (i, 0)),
        out_shape=jax.ShapeDtypeStruct((N, D), jnp.float32),
        scratch_shapes=[
            pltpu.VMEM((QBLK, N), jnp.float32),
            pltpu.VMEM((K, QBLK, 8), jnp.float32),
        ],
    )(xsgs, xsgs, xsT, W0p, enc_b0.reshape(1, NH), enc_W1,
      enc_b1.reshape(1, NH), enc_W2, enc_b2.reshape(1, NH),
      dec_W0, dec_W1, dec_W2)
    return out


# speedup vs baseline: 11.2239x; 1.6434x over previous
import functools

import jax
import jax.numpy as jnp
from jax import lax
from jax.experimental import pallas as pl
from jax.experimental.pallas import tpu as pltpu
from jax.experimental.pallas import tpu_sc as plsc

D = 3
N = 4096
K = 32
NH = 64
WIDTH = 1.0
NK = N * K

_info = plsc.get_sparse_core_info()
NC, NS, L = _info.num_cores, _info.num_subcores, _info.num_lanes
NW = NC * NS
QPW = N // NW
CPW = QPW * K

TAU1 = 0.024
BUFSZ = N + 48


def _sc_body(ptsT_hbm, out_hbm,
             px_v, py_v, pz_v, gx_v, gy_v, gz_v,
             bufd_v, bufi_v,
             s0_v, s1_v, s2_v, s3_v, s4_v, s5_v):
    wid = lax.axis_index("s") * NC + lax.axis_index("c")
    qbase = wid * QPW
    pts = (px_v, py_v, pz_v, gx_v, gy_v, gz_v)
    stage = (s0_v, s1_v, s2_v, s3_v, s4_v, s5_v)
    for d in range(6):
        pltpu.sync_copy(ptsT_hbm.at[d], pts[d])

    iota = lax.iota(jnp.int32, L)
    infv = jnp.full((L,), jnp.inf, jnp.float32)
    zero_i = jnp.zeros((L,), jnp.int32)

    def per_query(q, _):
        qi = qbase + q
        qsplat = jnp.full((L,), qi, jnp.int32)
        qx = plsc.load_gather(px_v, [qsplat])
        qy = plsc.load_gather(py_v, [qsplat])
        qz = plsc.load_gather(pz_v, [qsplat])
        qgx = plsc.load_gather(gx_v, [qsplat])
        qgy = plsc.load_gather(gy_v, [qsplat])
        qgz = plsc.load_gather(gz_v, [qsplat])

        def collect(tau):
            def cbody(j, wp):
                off = j * L
                ax = jnp.abs(qx - px_v[pl.ds(off, L)])
                ay = jnp.abs(qy - py_v[pl.ds(off, L)])
                az = jnp.abs(qz - pz_v[pl.ds(off, L)])
                wx = jnp.minimum(ax, 1.0 - ax)
                wy = jnp.minimum(ay, 1.0 - ay)
                wz = jnp.minimum(az, 1.0 - az)
                d2 = wx * wx + wy * wy + wz * wz
                msk = d2 < tau
                plsc.store_compressed(bufd_v.at[pl.ds(wp, L)], d2, mask=msk)
                plsc.store_compressed(bufi_v.at[pl.ds(wp, L)], iota + off,
                                      mask=msk)
                return wp + jnp.sum(msk.astype(jnp.int32))

            return lax.fori_loop(0, N // L, cbody, 0)

        wp = collect(TAU1)
        wp = lax.cond(wp < K, lambda: collect(4.0), lambda: wp)

        bufd_v[pl.ds(wp, L)] = infv
        bufd_v[pl.ds(wp + L, L)] = infv

        def ins(jc, st):
            s0k, s0v, s1k, s1v = st
            ck = bufd_v[pl.ds(jc * L, L)]
            cv = bufi_v[pl.ds(jc * L, L)]
            cks, cvs = plsc.sort_key_val(ck, cv)
            rbk = lax.rev(cks, (0,))
            rbv = lax.rev(cvs, (0,))
            m = s1k <= rbk
            lok = jnp.where(m, s1k, rbk)
            lov = jnp.where(m, s1v, rbv)
            lks, lvs = plsc.sort_key_val(lok, lov)
            rlk = lax.rev(lks, (0,))
            rlv = lax.rev(lvs, (0,))
            m2 = s0k <= rlk
            n0k = jnp.where(m2, s0k, rlk)
            n0v = jnp.where(m2, s0v, rlv)
            h1k = jnp.where(m2, rlk, s0k)
            h1v = jnp.where(m2, rlv, s0v)
            s0k, s0v = plsc.sort_key_val(n0k, n0v)
            s1k, s1v = plsc.sort_key_val(h1k, h1v)
            return (s0k, s0v, s1k, s1v)

        nch = (wp + L - 1) // L
        _, s0v_, _, s1v_ = lax.fori_loop(
            0, nch, ins, (infv, zero_i, infv, zero_i))

        colA = iota * QPW + q
        colB = colA + (K // 2) * QPW
        for d in range(3):
            qd = (qx, qy, qz)[d]
            for sv, col in ((s0v_, colA), (s1v_, colB)):
                nb = plsc.load_gather(pts[d], [sv])
                df = qd - nb
                w = (df - jnp.where(df > 0.5, 1.0, 0.0)
                        + jnp.where(df < -0.5, 1.0, 0.0))
                plsc.store_scatter(stage[d], [col], w)
        for d in range(3):
            qd = (qgx, qgy, qgz)[d]
            for sv, col in ((s0v_, colA), (s1v_, colB)):
                nb = plsc.load_gather(pts[3 + d], [sv])
                plsc.store_scatter(stage[3 + d], [col], qd - nb)
        return 0

    lax.fori_loop(0, QPW, per_query, 0)

    for d in range(6):
        pltpu.sync_copy(stage[d],
                        out_hbm.at[pl.ds(d * NK + wid * CPW, CPW)])


def _sc_knn(ptsT):
    kfn = pl.kernel(
        _sc_body,
        mesh=plsc.VectorSubcoreMesh(core_axis_name="c", subcore_axis_name="s"),
        out_type=jax.ShapeDtypeStruct((6 * NK,), jnp.float32),
        compiler_params=pltpu.CompilerParams(needs_layout_passes=False),
        scratch_types=(
            [pltpu.VMEM((N,), jnp.float32) for _ in range(6)]
            + [pltpu.VMEM((BUFSZ,), jnp.float32),
               pltpu.VMEM((BUFSZ,), jnp.int32)]
            + [pltpu.VMEM((CPW,), jnp.float32) for _ in range(6)]
        ),
    )
    return kfn(ptsT)


CB = 8192
WPB = CB // CPW


def _mlp_body(inpT_ref, w0_ref, b0_ref, w1_ref, b1_ref, w2_ref, b2_ref,
              d0_ref, d1_ref, d2_ref, outT_ref):
    mm = functools.partial(lax.dot_general,
                           dimension_numbers=(((1,), (0,)), ((), ())),
                           preferred_element_type=jnp.float32)
    gelu = functools.partial(jax.nn.gelu, approximate=True)
    blk = inpT_ref[...]
    h = gelu(mm(w0_ref[...], blk) + b0_ref[...])
    h = gelu(mm(w1_ref[...], h) + b1_ref[...])
    h = mm(w2_ref[...], h)
    pools = []
    for wb in range(WPB):
        acc = h[:, wb * CPW: wb * CPW + QPW]
        for k in range(1, K):
            acc = acc + h[:, wb * CPW + k * QPW: wb * CPW + (k + 1) * QPW]
        pools.append(acc)
    pooled = jnp.concatenate(pools, axis=1) * (1.0 / K) + b2_ref[...]
    g = gelu(mm(d0_ref[...], pooled))
    g = gelu(mm(d1_ref[...], g))
    outT_ref[...] = mm(d2_ref[...], g)


def _tc_mlp(inpT, w0T, b0c, w1T, b1c, w2T, b2c, d0T, d1T, d2T):
    full = lambda shape: pl.BlockSpec(shape, lambda i: (0,) * len(shape))
    return pl.pallas_call(
        _mlp_body,
        grid=(NK // CB,),
        in_specs=[
            pl.BlockSpec((6, CB), lambda i: (0, i)),
            full((NH, 6)), full((NH, 1)),
            full((NH, NH)), full((NH, 1)),
            full((NH, NH)), full((NH, 1)),
            full((NH, NH)), full((NH, NH)), full((D, NH)),
        ],
        out_specs=pl.BlockSpec((D, CB // K), lambda i: (0, i)),
        out_shape=jax.ShapeDtypeStruct((D, N), jnp.float32),
    )(inpT, w0T, b0c, w1T, b1c, w2T, b2c, d0T, d1T, d2T)


def kernel(xs, gs, enc_W0, enc_b0, enc_W1, enc_b1, enc_W2, enc_b2,
           dec_W0, dec_W1, dec_W2):
    ptsT = jnp.concatenate([xs.T, gs.T], axis=0)
    inpT = _sc_knn(ptsT).reshape(6, NK)
    outT = _tc_mlp(inpT,
                   enc_W0.T, enc_b0.reshape(NH, 1),
                   enc_W1.T, enc_b1.reshape(NH, 1),
                   enc_W2.T, enc_b2.reshape(NH, 1),
                   dec_W0.T, dec_W1.T, dec_W2.T)
    return outT.T
